# baseline (speedup 1.0000x reference)
import functools

import jax
import jax.numpy as jnp
from jax import lax
from jax.experimental import pallas as pl
from jax.experimental.pallas import tpu as pltpu
from jax.experimental.pallas import tpu_sc as plsc

N_NODES = 10000
D = 128
E = 320000

NC = 2
NS = 16
NW = NC * NS
EPW = E // NW
K = 128
NCHUNK = EPW // K
KTAIL = EPW - NCHUNK * K
CP = 80
NBLK = N_NODES // CP


def _relu_linear_body(x_ref, w_ref, b_ref, o_ref):
    acc = jnp.dot(x_ref[...], w_ref[...], preferred_element_type=jnp.float32)
    o_ref[...] = jnp.maximum(acc + b_ref[...], 0.0)


def _relu_linear(x, w1t, b1):
    nb = 10
    rb = N_NODES // nb
    return pl.pallas_call(
        _relu_linear_body,
        grid=(nb,),
        in_specs=[
            pl.BlockSpec((rb, D), lambda i: (i, 0)),
            pl.BlockSpec((D, D), lambda i: (0, 0)),
            pl.BlockSpec((1, D), lambda i: (0, 0)),
        ],
        out_specs=pl.BlockSpec((rb, D), lambda i: (i, 0)),
        out_shape=jax.ShapeDtypeStruct((N_NODES, D), jnp.float32),
    )(x, w1t, b1)


def _combine_body(p_ref, w_ref, b_ref, o_ref):
    s = p_ref[0] + p_ref[1]
    acc = jnp.dot(s, w_ref[...], preferred_element_type=jnp.float32)
    o_ref[...] = acc + b_ref[...]


def _combine(partials, w2t, b2):
    nb = 10
    rb = N_NODES // nb
    return pl.pallas_call(
        _combine_body,
        grid=(nb,),
        in_specs=[
            pl.BlockSpec((NC, rb, D), lambda i: (0, i, 0)),
            pl.BlockSpec((D, D), lambda i: (0, 0)),
            pl.BlockSpec((1, D), lambda i: (0, 0)),
        ],
        out_specs=pl.BlockSpec((rb, D), lambda i: (i, 0)),
        out_shape=jax.ShapeDtypeStruct((N_NODES, D), jnp.float32),
    )(partials, w2t, b2)


def _scatter_body(h_hbm, src_hbm, dst_hbm, out_hbm, acc_sh,
                  src_v0, src_v1, src_v2, src_v3,
                  dst_v0, dst_v1, dst_v2, dst_v3,
                  rows_v, stage_v, tsrc_v, tdst_v, trows_v,
                  isem0, isem1, isem2, isem3, gsem0, gsem1, ssem0, ssem1):
    src_v = [src_v0, src_v1, src_v2, src_v3]
    dst_v = [dst_v0, dst_v1, dst_v2, dst_v3]
    isem = [isem0, isem1, isem2, isem3]
    gsem = [gsem0, gsem1]
    ssem = [ssem0, ssem1]
    cid = lax.axis_index("c")
    sid = lax.axis_index("s")
    wid = sid * NC + cid

    def _zero_row(i, _):
        for c in range(D // 16):
            stage_v[i, pl.ds(c * 16, 16)] = jnp.zeros((16,), jnp.float32)
        return 0
    lax.fori_loop(0, CP, _zero_row, 0)
    for j in range((NBLK + NS - 1) // NS):
        blk = j * NS + sid
        @pl.when(blk < NBLK)
        def _():
            pltpu.sync_copy(stage_v, acc_sh.at[pl.ds(blk * CP, CP)])
    plsc.subcore_barrier()

    ebase = wid * EPW

    def _start_idx(ci, q):
        off = ebase + ci * K
        pltpu.async_copy(src_hbm.at[pl.ds(off, K)], src_v[q], isem[q])
        pltpu.async_copy(dst_hbm.at[pl.ds(off, K)], dst_v[q], isem[q])

    def _wait_idx(q):
        pltpu.make_async_copy(src_hbm.at[pl.ds(0, K)], src_v[q],
                              isem[q]).wait()
        pltpu.make_async_copy(dst_hbm.at[pl.ds(0, K)], dst_v[q],
                              isem[q]).wait()

    def _start_gather(b, q):
        pltpu.async_copy(h_hbm.at[src_v[q]], rows_v.at[b], gsem[b])

    def _wait_gather(b):
        pltpu.make_async_copy(h_hbm.at[pl.ds(0, K)], rows_v.at[b],
                              gsem[b]).wait()

    def _start_scatter(b, q):
        pltpu.async_copy(rows_v.at[b], acc_sh.at[dst_v[q]], ssem[b],
                         add=True)

    def _wait_scatter(b):
        pltpu.make_async_copy(rows_v.at[b], acc_sh.at[dst_v[0]],
                              ssem[b]).wait()

    _start_idx(0, 0)
    _start_idx(1, 1)
    _start_idx(2, 2)
    _wait_idx(0)
    _start_gather(0, 0)

    def _quad(Q, _):
        for u in range(4):
            ci = Q * 4 + u
            b, nxt, q = u % 2, 1 - u % 2, u
            _wait_gather(b)
            _start_scatter(b, q)
            _wait_idx((q + 1) % 4)
            if u == 0:
                @pl.when(Q >= 1)
                def _():
                    _wait_scatter(nxt)
            else:
                _wait_scatter(nxt)
            _start_gather(nxt, (q + 1) % 4)

            @pl.when(ci + 3 < NCHUNK)
            def _():
                _start_idx(ci + 3, (q + 3) % 4)
        return 0
    lax.fori_loop(0, (NCHUNK - 2) // 4, _quad, 0)

    _wait_gather(0)
    _start_scatter(0, (NCHUNK - 2) % 4)
    _wait_idx((NCHUNK - 1) % 4)
    _wait_scatter(1)
    _start_gather(1, (NCHUNK - 1) % 4)
    _wait_gather(1)
    _start_scatter(1, (NCHUNK - 1) % 4)
    _wait_scatter(0)
    _wait_scatter(1)

    if KTAIL:
        toff = ebase + NCHUNK * K
        pltpu.sync_copy(src_hbm.at[pl.ds(toff, KTAIL)], tsrc_v)
        pltpu.sync_copy(dst_hbm.at[pl.ds(toff, KTAIL)], tdst_v)
        pltpu.async_copy(h_hbm.at[tsrc_v], trows_v, gsem[0]).wait()
        pltpu.sync_copy(trows_v, acc_sh.at[tdst_v], add=True)
    plsc.subcore_barrier()

    for j in range((NBLK + NS - 1) // NS):
        blk = j * NS + sid
        @pl.when(blk < NBLK)
        def _():
            r0 = blk * CP
            pltpu.sync_copy(acc_sh.at[pl.ds(r0, CP)], stage_v)
            pltpu.sync_copy(stage_v, out_hbm.at[cid, pl.ds(r0, CP)])


def _edge_scatter(h, src, dst):
    mesh = plsc.VectorSubcoreMesh(core_axis_name="c", subcore_axis_name="s")
    kern = pl.kernel(
        _scatter_body,
        out_type=jax.ShapeDtypeStruct((NC, N_NODES, D), jnp.float32),
        mesh=mesh,
        scratch_types=[
            pltpu.VMEM_SHARED((N_NODES, D), jnp.float32),
            pltpu.VMEM((K,), jnp.int32),
            pltpu.VMEM((K,), jnp.int32),
            pltpu.VMEM((K,), jnp.int32),
            pltpu.VMEM((K,), jnp.int32),
            pltpu.VMEM((K,), jnp.int32),
            pltpu.VMEM((K,), jnp.int32),
            pltpu.VMEM((K,), jnp.int32),
            pltpu.VMEM((K,), jnp.int32),
            pltpu.VMEM((2, K, D), jnp.float32),
            pltpu.VMEM((CP, D), jnp.float32),
            pltpu.VMEM((KTAIL,), jnp.int32),
            pltpu.VMEM((KTAIL,), jnp.int32),
            pltpu.VMEM((KTAIL, D), jnp.float32),
        ] + [pltpu.SemaphoreType.DMA] * 8,
    )
    return kern(h, src, dst)


def kernel(node_feats, edge_index, W1, b1, W2, b2):
    w1t = W1.T
    w2t = W2.T
    b1r = b1.reshape(1, D)
    b2r = b2.reshape(1, D)
    h = _relu_linear(node_feats, w1t, b1r)
    partials = _edge_scatter(h, edge_index[0], edge_index[1])
    return _combine(partials, w2t, b2r)


# speedup vs baseline: 1.0746x; 1.0746x over previous
import functools

import jax
import jax.numpy as jnp
from jax import lax
from jax.experimental import pallas as pl
from jax.experimental.pallas import tpu as pltpu
from jax.experimental.pallas import tpu_sc as plsc

N_NODES = 10000
D = 128
E = 320000

NC = 2
NS = 16
NW = NC * NS
EPW = E // NW
K = 128
NCHUNK = EPW // K
KTAIL = EPW - NCHUNK * K
CP = 80
NBLK = N_NODES // CP


def _relu_linear_body(x_ref, w_ref, b_ref, o_ref):
    acc = lax.dot_general(x_ref[...], w_ref[...], (((1,), (1,)), ((), ())),
                          preferred_element_type=jnp.float32)
    o_ref[...] = jnp.maximum(acc + b_ref[...], 0.0)


def _relu_linear(x, w1t, b1):
    nb = 10
    rb = N_NODES // nb
    return pl.pallas_call(
        _relu_linear_body,
        grid=(nb,),
        in_specs=[
            pl.BlockSpec((rb, D), lambda i: (i, 0)),
            pl.BlockSpec((D, D), lambda i: (0, 0)),
            pl.BlockSpec((1, D), lambda i: (0, 0)),
        ],
        out_specs=pl.BlockSpec((rb, D), lambda i: (i, 0)),
        out_shape=jax.ShapeDtypeStruct((N_NODES, D), jnp.float32),
    )(x, w1t, b1)


def _combine_body(p_ref, w_ref, b_ref, o_ref):
    s = p_ref[0] + p_ref[1]
    acc = lax.dot_general(s, w_ref[...], (((1,), (1,)), ((), ())),
                          preferred_element_type=jnp.float32)
    o_ref[...] = acc + b_ref[...]


def _combine(partials, w2t, b2):
    nb = 10
    rb = N_NODES // nb
    return pl.pallas_call(
        _combine_body,
        grid=(nb,),
        in_specs=[
            pl.BlockSpec((NC, rb, D), lambda i: (0, i, 0)),
            pl.BlockSpec((D, D), lambda i: (0, 0)),
            pl.BlockSpec((1, D), lambda i: (0, 0)),
        ],
        out_specs=pl.BlockSpec((rb, D), lambda i: (i, 0)),
        out_shape=jax.ShapeDtypeStruct((N_NODES, D), jnp.float32),
    )(partials, w2t, b2)


def _scatter_body(h_hbm, ei_hbm, out_hbm, acc_sh,
                  src_v0, src_v1, src_v2, src_v3,
                  dst_v0, dst_v1, dst_v2, dst_v3,
                  rows_v, stage_v, tsrc_v, tdst_v, trows_v,
                  isem0, isem1, isem2, isem3, gsem0, gsem1, ssem0, ssem1):
    src_v = [src_v0, src_v1, src_v2, src_v3]
    dst_v = [dst_v0, dst_v1, dst_v2, dst_v3]
    isem = [isem0, isem1, isem2, isem3]
    gsem = [gsem0, gsem1]
    ssem = [ssem0, ssem1]
    cid = lax.axis_index("c")
    sid = lax.axis_index("s")
    wid = sid * NC + cid

    ebase = wid * EPW

    def _start_idx(ci, q):
        off = ebase + ci * K
        pltpu.async_copy(ei_hbm.at[pl.ds(off, K)], src_v[q], isem[q])
        pltpu.async_copy(ei_hbm.at[pl.ds(E + off, K)], dst_v[q], isem[q])

    def _wait_idx(q):
        pltpu.make_async_copy(ei_hbm.at[pl.ds(0, K)], src_v[q],
                              isem[q]).wait()
        pltpu.make_async_copy(ei_hbm.at[pl.ds(0, K)], dst_v[q],
                              isem[q]).wait()

    def _start_gather(b, q):
        pltpu.async_copy(h_hbm.at[src_v[q]], rows_v.at[b], gsem[b])

    def _wait_gather(b):
        pltpu.make_async_copy(h_hbm.at[pl.ds(0, K)], rows_v.at[b],
                              gsem[b]).wait()

    def _start_scatter(b, q):
        pltpu.async_copy(rows_v.at[b], acc_sh.at[dst_v[q]], ssem[b],
                         add=True)

    def _wait_scatter(b):
        pltpu.make_async_copy(rows_v.at[b], acc_sh.at[dst_v[0]],
                              ssem[b]).wait()

    _start_idx(0, 0)
    _start_idx(1, 1)
    _start_idx(2, 2)

    def _zero_row(i, _):
        for c in range(D // 16):
            stage_v[i, pl.ds(c * 16, 16)] = jnp.zeros((16,), jnp.float32)
        return 0
    lax.fori_loop(0, CP, _zero_row, 0)
    _wait_idx(0)
    _start_gather(0, 0)
    for j in range((NBLK + NS - 1) // NS):
        blk = j * NS + sid
        @pl.when(blk < NBLK)
        def _():
            pltpu.sync_copy(stage_v, acc_sh.at[pl.ds(blk * CP, CP)])
    plsc.subcore_barrier()

    def _quad(Q, _):
        for u in range(4):
            ci = Q * 4 + u
            b, nxt, q = u % 2, 1 - u % 2, u
            _wait_gather(b)
            _start_scatter(b, q)
            _wait_idx((q + 1) % 4)
            if u == 0:
                @pl.when(Q >= 1)
                def _():
                    _wait_scatter(nxt)
            else:
                _wait_scatter(nxt)
            _start_gather(nxt, (q + 1) % 4)

            @pl.when(ci + 3 < NCHUNK)
            def _():
                _start_idx(ci + 3, (q + 3) % 4)
        return 0
    lax.fori_loop(0, (NCHUNK - 2) // 4, _quad, 0)

    _wait_gather(0)
    _start_scatter(0, (NCHUNK - 2) % 4)
    _wait_idx((NCHUNK - 1) % 4)
    _wait_scatter(1)
    _start_gather(1, (NCHUNK - 1) % 4)
    _wait_gather(1)
    _start_scatter(1, (NCHUNK - 1) % 4)
    _wait_scatter(0)
    _wait_scatter(1)

    if KTAIL:
        toff = ebase + NCHUNK * K
        pltpu.sync_copy(ei_hbm.at[pl.ds(toff, KTAIL)], tsrc_v)
        pltpu.sync_copy(ei_hbm.at[pl.ds(E + toff, KTAIL)], tdst_v)
        pltpu.async_copy(h_hbm.at[tsrc_v], trows_v, gsem[0]).wait()
        pltpu.sync_copy(trows_v, acc_sh.at[tdst_v], add=True)
    plsc.subcore_barrier()

    for j in range((NBLK + NS - 1) // NS):
        blk = j * NS + sid
        @pl.when(blk < NBLK)
        def _():
            r0 = blk * CP
            pltpu.sync_copy(acc_sh.at[pl.ds(r0, CP)], stage_v)
            pltpu.sync_copy(stage_v, out_hbm.at[cid, pl.ds(r0, CP)])


def _edge_scatter(h, ei_flat):
    mesh = plsc.VectorSubcoreMesh(core_axis_name="c", subcore_axis_name="s")
    kern = pl.kernel(
        _scatter_body,
        out_type=jax.ShapeDtypeStruct((NC, N_NODES, D), jnp.float32),
        mesh=mesh,
        scratch_types=[
            pltpu.VMEM_SHARED((N_NODES, D), jnp.float32),
            pltpu.VMEM((K,), jnp.int32),
            pltpu.VMEM((K,), jnp.int32),
            pltpu.VMEM((K,), jnp.int32),
            pltpu.VMEM((K,), jnp.int32),
            pltpu.VMEM((K,), jnp.int32),
            pltpu.VMEM((K,), jnp.int32),
            pltpu.VMEM((K,), jnp.int32),
            pltpu.VMEM((K,), jnp.int32),
            pltpu.VMEM((2, K, D), jnp.float32),
            pltpu.VMEM((CP, D), jnp.float32),
            pltpu.VMEM((KTAIL,), jnp.int32),
            pltpu.VMEM((KTAIL,), jnp.int32),
            pltpu.VMEM((KTAIL, D), jnp.float32),
        ] + [pltpu.SemaphoreType.DMA] * 8,
    )
    return kern(h, ei_flat)


def kernel(node_feats, edge_index, W1, b1, W2, b2):
    b1r = b1.reshape(1, D)
    b2r = b2.reshape(1, D)
    h = _relu_linear(node_feats, W1, b1r)
    partials = _edge_scatter(h, edge_index.reshape(2 * E))
    return _combine(partials, W2, b2r)


# speedup vs baseline: 1.0774x; 1.0026x over previous
import functools

import jax
import jax.numpy as jnp
from jax import lax
from jax.experimental import pallas as pl
from jax.experimental.pallas import tpu as pltpu
from jax.experimental.pallas import tpu_sc as plsc

N_NODES = 10000
D = 128
E = 320000

NC = 2
NS = 16
NW = NC * NS
EPW = E // NW
K = 128
NCHUNK = EPW // K
KTAIL = EPW - NCHUNK * K
CP = 80
NBLK = N_NODES // CP


def _relu_linear_body(x_ref, w_ref, b_ref, o_ref):
    acc = lax.dot_general(x_ref[...], w_ref[...], (((1,), (1,)), ((), ())),
                          preferred_element_type=jnp.float32)
    o_ref[...] = jnp.maximum(acc + b_ref[...], 0.0)


def _relu_linear(x, w1t, b1):
    nb = 10
    rb = N_NODES // nb
    return pl.pallas_call(
        _relu_linear_body,
        grid=(nb,),
        in_specs=[
            pl.BlockSpec((rb, D), lambda i: (i, 0)),
            pl.BlockSpec((D, D), lambda i: (0, 0)),
            pl.BlockSpec((1, D), lambda i: (0, 0)),
        ],
        out_specs=pl.BlockSpec((rb, D), lambda i: (i, 0)),
        out_shape=jax.ShapeDtypeStruct((N_NODES, D), jnp.float32),
    )(x, w1t, b1)


def _combine_body(p_ref, w_ref, b_ref, o_ref):
    s = p_ref[0] + p_ref[1]
    acc = lax.dot_general(s, w_ref[...], (((1,), (1,)), ((), ())),
                          preferred_element_type=jnp.float32)
    o_ref[...] = acc + b_ref[...]


def _combine(partials, w2t, b2):
    nb = 10
    rb = N_NODES // nb
    return pl.pallas_call(
        _combine_body,
        grid=(nb,),
        in_specs=[
            pl.BlockSpec((NC, rb, D), lambda i: (0, i, 0)),
            pl.BlockSpec((D, D), lambda i: (0, 0)),
            pl.BlockSpec((1, D), lambda i: (0, 0)),
        ],
        out_specs=pl.BlockSpec((rb, D), lambda i: (i, 0)),
        out_shape=jax.ShapeDtypeStruct((N_NODES, D), jnp.float32),
    )(partials, w2t, b2)


def _scatter_body(h_hbm, ei_hbm, out_hbm, acc_sh,
                  src_v0, src_v1, src_v2, src_v3,
                  dst_v0, dst_v1, dst_v2, dst_v3,
                  rows_v, stage_v, tsrc_v, tdst_v, trows_v,
                  isem0, isem1, isem2, isem3, gsem0, gsem1, ssem0, ssem1):
    src_v = [src_v0, src_v1, src_v2, src_v3]
    dst_v = [dst_v0, dst_v1, dst_v2, dst_v3]
    isem = [isem0, isem1, isem2, isem3]
    gsem = [gsem0, gsem1]
    ssem = [ssem0, ssem1]
    cid = lax.axis_index("c")
    sid = lax.axis_index("s")
    wid = sid * NC + cid

    ebase = wid * EPW

    def _start_idx(ci, q):
        off = ebase + ci * K
        pltpu.async_copy(ei_hbm.at[pl.ds(off, K)], src_v[q], isem[q])
        pltpu.async_copy(ei_hbm.at[pl.ds(E + off, K)], dst_v[q], isem[q])

    def _wait_idx(q):
        pltpu.make_async_copy(ei_hbm.at[pl.ds(0, K)], src_v[q],
                              isem[q]).wait()
        pltpu.make_async_copy(ei_hbm.at[pl.ds(0, K)], dst_v[q],
                              isem[q]).wait()

    KH = K // 2

    def _start_gather(b, q):
        pltpu.async_copy(h_hbm.at[src_v[q].at[pl.ds(0, KH)]],
                         rows_v.at[b, pl.ds(0, KH)], gsem[b])
        pltpu.async_copy(h_hbm.at[src_v[q].at[pl.ds(KH, KH)]],
                         rows_v.at[b, pl.ds(KH, KH)], gsem[b])

    def _wait_gather(b):
        pltpu.make_async_copy(h_hbm.at[pl.ds(0, KH)],
                              rows_v.at[b, pl.ds(0, KH)], gsem[b]).wait()
        pltpu.make_async_copy(h_hbm.at[pl.ds(0, KH)],
                              rows_v.at[b, pl.ds(KH, KH)], gsem[b]).wait()

    def _start_scatter(b, q):
        pltpu.async_copy(rows_v.at[b], acc_sh.at[dst_v[q]], ssem[b],
                         add=True)

    def _wait_scatter(b):
        pltpu.make_async_copy(rows_v.at[b], acc_sh.at[dst_v[0]],
                              ssem[b]).wait()

    _start_idx(0, 0)
    _start_idx(1, 1)
    _start_idx(2, 2)

    def _zero_row(i, _):
        for c in range(D // 16):
            stage_v[i, pl.ds(c * 16, 16)] = jnp.zeros((16,), jnp.float32)
        return 0
    lax.fori_loop(0, CP, _zero_row, 0)
    _wait_idx(0)
    _start_gather(0, 0)
    for j in range((NBLK + NS - 1) // NS):
        blk = j * NS + sid
        @pl.when(blk < NBLK)
        def _():
            pltpu.sync_copy(stage_v, acc_sh.at[pl.ds(blk * CP, CP)])
    plsc.subcore_barrier()

    def _quad(Q, _):
        for u in range(4):
            ci = Q * 4 + u
            b, nxt, q = u % 2, 1 - u % 2, u
            _wait_gather(b)
            _start_scatter(b, q)
            _wait_idx((q + 1) % 4)
            if u == 0:
                @pl.when(Q >= 1)
                def _():
                    _wait_scatter(nxt)
            else:
                _wait_scatter(nxt)
            _start_gather(nxt, (q + 1) % 4)

            @pl.when(ci + 3 < NCHUNK)
            def _():
                _start_idx(ci + 3, (q + 3) % 4)
        return 0
    lax.fori_loop(0, (NCHUNK - 2) // 4, _quad, 0)

    _wait_gather(0)
    _start_scatter(0, (NCHUNK - 2) % 4)
    _wait_idx((NCHUNK - 1) % 4)
    _wait_scatter(1)
    _start_gather(1, (NCHUNK - 1) % 4)
    _wait_gather(1)
    _start_scatter(1, (NCHUNK - 1) % 4)
    _wait_scatter(0)
    _wait_scatter(1)

    if KTAIL:
        toff = ebase + NCHUNK * K
        pltpu.sync_copy(ei_hbm.at[pl.ds(toff, KTAIL)], tsrc_v)
        pltpu.sync_copy(ei_hbm.at[pl.ds(E + toff, KTAIL)], tdst_v)
        pltpu.async_copy(h_hbm.at[tsrc_v], trows_v, gsem[0]).wait()
        pltpu.sync_copy(trows_v, acc_sh.at[tdst_v], add=True)
    plsc.subcore_barrier()

    for j in range((NBLK + NS - 1) // NS):
        blk = j * NS + sid
        @pl.when(blk < NBLK)
        def _():
            r0 = blk * CP
            pltpu.sync_copy(acc_sh.at[pl.ds(r0, CP)], stage_v)
            pltpu.sync_copy(stage_v, out_hbm.at[cid, pl.ds(r0, CP)])


def _edge_scatter(h, ei_flat):
    mesh = plsc.VectorSubcoreMesh(core_axis_name="c", subcore_axis_name="s")
    kern = pl.kernel(
        _scatter_body,
        out_type=jax.ShapeDtypeStruct((NC, N_NODES, D), jnp.float32),
        mesh=mesh,
        scratch_types=[
            pltpu.VMEM_SHARED((N_NODES, D), jnp.float32),
            pltpu.VMEM((K,), jnp.int32),
            pltpu.VMEM((K,), jnp.int32),
            pltpu.VMEM((K,), jnp.int32),
            pltpu.VMEM((K,), jnp.int32),
            pltpu.VMEM((K,), jnp.int32),
            pltpu.VMEM((K,), jnp.int32),
            pltpu.VMEM((K,), jnp.int32),
            pltpu.VMEM((K,), jnp.int32),
            pltpu.VMEM((2, K, D), jnp.float32),
            pltpu.VMEM((CP, D), jnp.float32),
            pltpu.VMEM((KTAIL,), jnp.int32),
            pltpu.VMEM((KTAIL,), jnp.int32),
            pltpu.VMEM((KTAIL, D), jnp.float32),
        ] + [pltpu.SemaphoreType.DMA] * 8,
    )
    return kern(h, ei_flat)


def kernel(node_feats, edge_index, W1, b1, W2, b2):
    b1r = b1.reshape(1, D)
    b2r = b2.reshape(1, D)
    h = _relu_linear(node_feats, W1, b1r)
    partials = _edge_scatter(h, edge_index.reshape(2 * E))
    return _combine(partials, W2, b2r)


# speedup vs baseline: 1.0967x; 1.0179x over previous
import functools

import jax
import jax.numpy as jnp
from jax import lax
from jax.experimental import pallas as pl
from jax.experimental.pallas import tpu as pltpu
from jax.experimental.pallas import tpu_sc as plsc

N_NODES = 10000
D = 128
E = 320000

NC = 2
NS = 16
NW = NC * NS
EPW = E // NW
K = 128
NCHUNK = EPW // K
KTAIL = EPW - NCHUNK * K
CP = 80
NBLK = N_NODES // CP


def _relu_linear_body(x_ref, w_ref, b_ref, o_ref):
    acc = lax.dot_general(x_ref[...], w_ref[...], (((1,), (1,)), ((), ())),
                          preferred_element_type=jnp.float32)
    o_ref[...] = jnp.maximum(acc + b_ref[...], 0.0)


def _relu_linear(x, w1t, b1):
    nb = 10
    rb = N_NODES // nb
    return pl.pallas_call(
        _relu_linear_body,
        grid=(nb,),
        in_specs=[
            pl.BlockSpec((rb, D), lambda i: (i, 0)),
            pl.BlockSpec((D, D), lambda i: (0, 0)),
            pl.BlockSpec((1, D), lambda i: (0, 0)),
        ],
        out_specs=pl.BlockSpec((rb, D), lambda i: (i, 0)),
        out_shape=jax.ShapeDtypeStruct((N_NODES, D), jnp.float32),
    )(x, w1t, b1)


def _combine_body(p_ref, w_ref, b_ref, o_ref):
    s = p_ref[0] + p_ref[1]
    acc = lax.dot_general(s, w_ref[...], (((1,), (1,)), ((), ())),
                          preferred_element_type=jnp.float32)
    o_ref[...] = acc + b_ref[...]


def _combine(partials, w2t, b2):
    nb = 10
    rb = N_NODES // nb
    return pl.pallas_call(
        _combine_body,
        grid=(nb,),
        in_specs=[
            pl.BlockSpec((NC, rb, D), lambda i: (0, i, 0)),
            pl.BlockSpec((D, D), lambda i: (0, 0)),
            pl.BlockSpec((1, D), lambda i: (0, 0)),
        ],
        out_specs=pl.BlockSpec((rb, D), lambda i: (i, 0)),
        out_shape=jax.ShapeDtypeStruct((N_NODES, D), jnp.float32),
    )(partials, w2t, b2)


def _scatter_body(h_hbm, ei_hbm, out_hbm, acc_sh,
                  src_v0, src_v1, src_v2, src_v3,
                  dst_v0, dst_v1, dst_v2, dst_v3,
                  rows_v, stage_v, tsrc_v, tdst_v, trows_v,
                  isem0, isem1, isem2, isem3, gsem0, gsem1, ssem0, ssem1,
                  zsem):
    src_v = [src_v0, src_v1, src_v2, src_v3]
    dst_v = [dst_v0, dst_v1, dst_v2, dst_v3]
    isem = [isem0, isem1, isem2, isem3]
    gsem = [gsem0, gsem1]
    ssem = [ssem0, ssem1]
    cid = lax.axis_index("c")
    sid = lax.axis_index("s")
    wid = sid * NC + cid

    ebase = wid * EPW

    def _start_idx(ci, q):
        off = ebase + ci * K
        pltpu.async_copy(ei_hbm.at[pl.ds(off, K)], src_v[q], isem[q])
        pltpu.async_copy(ei_hbm.at[pl.ds(E + off, K)], dst_v[q], isem[q])

    def _wait_idx(q):
        pltpu.make_async_copy(ei_hbm.at[pl.ds(0, K)], src_v[q],
                              isem[q]).wait()
        pltpu.make_async_copy(ei_hbm.at[pl.ds(0, K)], dst_v[q],
                              isem[q]).wait()

    KH = K // 2

    def _start_gather(b, q):
        pltpu.async_copy(h_hbm.at[src_v[q].at[pl.ds(0, KH)]],
                         rows_v.at[b, pl.ds(0, KH)], gsem[b])
        pltpu.async_copy(h_hbm.at[src_v[q].at[pl.ds(KH, KH)]],
                         rows_v.at[b, pl.ds(KH, KH)], gsem[b])

    def _wait_gather(b):
        pltpu.make_async_copy(h_hbm.at[pl.ds(0, KH)],
                              rows_v.at[b, pl.ds(0, KH)], gsem[b]).wait()
        pltpu.make_async_copy(h_hbm.at[pl.ds(0, KH)],
                              rows_v.at[b, pl.ds(KH, KH)], gsem[b]).wait()

    def _start_scatter(b, q):
        pltpu.async_copy(rows_v.at[b], acc_sh.at[dst_v[q]], ssem[b],
                         add=True)

    def _wait_scatter(b):
        pltpu.make_async_copy(rows_v.at[b], acc_sh.at[dst_v[0]],
                              ssem[b]).wait()

    _start_idx(0, 0)
    _start_idx(1, 1)
    _start_idx(2, 2)

    def _zero_row(i, _):
        for c in range(D // 16):
            stage_v[i, pl.ds(c * 16, 16)] = jnp.zeros((16,), jnp.float32)
        return 0
    lax.fori_loop(0, CP, _zero_row, 0)
    _wait_idx(0)
    _start_gather(0, 0)
    JB = (NBLK + NS - 1) // NS
    for j in range(JB):
        blk = j * NS + sid
        @pl.when(blk < NBLK)
        def _():
            pltpu.async_copy(stage_v, acc_sh.at[pl.ds(blk * CP, CP)], zsem)
    for j in range(JB):
        blk = j * NS + sid
        @pl.when(blk < NBLK)
        def _():
            pltpu.make_async_copy(stage_v, acc_sh.at[pl.ds(blk * CP, CP)],
                                  zsem).wait()
    plsc.subcore_barrier()

    def _quad(Q, _):
        for u in range(4):
            ci = Q * 4 + u
            b, nxt, q = u % 2, 1 - u % 2, u
            _wait_gather(b)
            _start_scatter(b, q)
            _wait_idx((q + 1) % 4)
            if u == 0:
                @pl.when(Q >= 1)
                def _():
                    _wait_scatter(nxt)
            else:
                _wait_scatter(nxt)
            _start_gather(nxt, (q + 1) % 4)

            @pl.when(ci + 3 < NCHUNK)
            def _():
                _start_idx(ci + 3, (q + 3) % 4)
        return 0
    lax.fori_loop(0, (NCHUNK - 2) // 4, _quad, 0)

    _wait_gather(0)
    _start_scatter(0, (NCHUNK - 2) % 4)
    _wait_idx((NCHUNK - 1) % 4)
    _wait_scatter(1)
    _start_gather(1, (NCHUNK - 1) % 4)
    _wait_gather(1)
    _start_scatter(1, (NCHUNK - 1) % 4)
    _wait_scatter(0)
    _wait_scatter(1)

    if KTAIL:
        toff = ebase + NCHUNK * K
        pltpu.sync_copy(ei_hbm.at[pl.ds(toff, KTAIL)], tsrc_v)
        pltpu.sync_copy(ei_hbm.at[pl.ds(E + toff, KTAIL)], tdst_v)
        pltpu.async_copy(h_hbm.at[tsrc_v], trows_v, gsem[0]).wait()
        pltpu.sync_copy(trows_v, acc_sh.at[tdst_v], add=True)
    plsc.subcore_barrier()

    for j in range(JB + 2):
        if j >= 2:
            blkw = (j - 2) * NS + sid
            @pl.when(blkw < NBLK)
            def _():
                pltpu.make_async_copy(
                    rows_v.at[j % 2, pl.ds(0, CP)],
                    out_hbm.at[cid, pl.ds(blkw * CP, CP)],
                    gsem[j % 2]).wait()
        if j < JB:
            blk = j * NS + sid
            @pl.when(blk < NBLK)
            def _():
                r0 = blk * CP
                pltpu.sync_copy(acc_sh.at[pl.ds(r0, CP)],
                                rows_v.at[j % 2, pl.ds(0, CP)])
                pltpu.async_copy(rows_v.at[j % 2, pl.ds(0, CP)],
                                 out_hbm.at[cid, pl.ds(r0, CP)],
                                 gsem[j % 2])


def _edge_scatter(h, ei_flat):
    mesh = plsc.VectorSubcoreMesh(core_axis_name="c", subcore_axis_name="s")
    kern = pl.kernel(
        _scatter_body,
        out_type=jax.ShapeDtypeStruct((NC, N_NODES, D), jnp.float32),
        mesh=mesh,
        scratch_types=[
            pltpu.VMEM_SHARED((N_NODES, D), jnp.float32),
            pltpu.VMEM((K,), jnp.int32),
            pltpu.VMEM((K,), jnp.int32),
            pltpu.VMEM((K,), jnp.int32),
            pltpu.VMEM((K,), jnp.int32),
            pltpu.VMEM((K,), jnp.int32),
            pltpu.VMEM((K,), jnp.int32),
            pltpu.VMEM((K,), jnp.int32),
            pltpu.VMEM((K,), jnp.int32),
            pltpu.VMEM((2, K, D), jnp.float32),
            pltpu.VMEM((CP, D), jnp.float32),
            pltpu.VMEM((KTAIL,), jnp.int32),
            pltpu.VMEM((KTAIL,), jnp.int32),
            pltpu.VMEM((KTAIL, D), jnp.float32),
        ] + [pltpu.SemaphoreType.DMA] * 9,
    )
    return kern(h, ei_flat)


def kernel(node_feats, edge_index, W1, b1, W2, b2):
    b1r = b1.reshape(1, D)
    b2r = b2.reshape(1, D)
    h = _relu_linear(node_feats, W1, b1r)
    partials = _edge_scatter(h, edge_index.reshape(2 * E))
    return _combine(partials, W2, b2r)


# speedup vs baseline: 1.1668x; 1.0640x over previous
import functools

import jax
import jax.numpy as jnp
from jax import lax
from jax.experimental import pallas as pl
from jax.experimental.pallas import tpu as pltpu
from jax.experimental.pallas import tpu_sc as plsc

N_NODES = 10000
D = 128
E = 320000

NC = 2
NS = 16
NW = NC * NS
K = 128
NCHTOT = E // K
NCHUNK = NCHTOT // NW
NXTRA = NCHTOT - NCHUNK * NW
CP = 80
NBLK = N_NODES // CP


def _relu_linear_body(x_ref, w_ref, b_ref, o_ref):
    acc = lax.dot_general(x_ref[...], w_ref[...], (((1,), (1,)), ((), ())),
                          preferred_element_type=jnp.float32)
    o_ref[...] = jnp.maximum(acc + b_ref[...], 0.0)


def _relu_linear(x, w1t, b1):
    nb = 5
    rb = N_NODES // nb
    return pl.pallas_call(
        _relu_linear_body,
        grid=(nb,),
        in_specs=[
            pl.BlockSpec((rb, D), lambda i: (i, 0)),
            pl.BlockSpec((D, D), lambda i: (0, 0)),
            pl.BlockSpec((1, D), lambda i: (0, 0)),
        ],
        out_specs=pl.BlockSpec((rb, D), lambda i: (i, 0)),
        out_shape=jax.ShapeDtypeStruct((N_NODES, D), jnp.float32),
    )(x, w1t, b1)


def _combine_body(p_ref, w_ref, b_ref, o_ref):
    s = p_ref[0] + p_ref[1]
    acc = lax.dot_general(s, w_ref[...], (((1,), (1,)), ((), ())),
                          preferred_element_type=jnp.float32)
    o_ref[...] = acc + b_ref[...]


def _combine(partials, w2t, b2):
    nb = 5
    rb = N_NODES // nb
    return pl.pallas_call(
        _combine_body,
        grid=(nb,),
        in_specs=[
            pl.BlockSpec((NC, rb, D), lambda i: (0, i, 0)),
            pl.BlockSpec((D, D), lambda i: (0, 0)),
            pl.BlockSpec((1, D), lambda i: (0, 0)),
        ],
        out_specs=pl.BlockSpec((rb, D), lambda i: (i, 0)),
        out_shape=jax.ShapeDtypeStruct((N_NODES, D), jnp.float32),
    )(partials, w2t, b2)


def _scatter_body(h_hbm, ei_hbm, out_hbm, acc_sh,
                  ei_v0, ei_v1, ei_v2, ei_v3, ei_vx,
                  rows_v, stage_v,
                  isem0, isem1, isem2, isem3, gsem0, gsem1, ssem0, ssem1,
                  zsem):
    ei_v = [ei_v0, ei_v1, ei_v2, ei_v3]
    isem = [isem0, isem1, isem2, isem3]
    gsem = [gsem0, gsem1]
    ssem = [ssem0, ssem1]
    cid = lax.axis_index("c")
    sid = lax.axis_index("s")
    wid = sid * NC + cid

    def _start_idx(ci, q):
        off = (wid + NW * ci) * K
        pltpu.async_copy(ei_hbm.at[:, pl.ds(off, K)], ei_v[q], isem[q])

    def _wait_idx(q):
        pltpu.make_async_copy(ei_hbm.at[:, pl.ds(0, K)], ei_v[q],
                              isem[q]).wait()

    KH = K // 2

    def _start_gather(b, q):
        src = ei_v[q].at[0]
        pltpu.async_copy(h_hbm.at[src.at[pl.ds(0, KH)]],
                         rows_v.at[b, pl.ds(0, KH)], gsem[b])
        pltpu.async_copy(h_hbm.at[src.at[pl.ds(KH, KH)]],
                         rows_v.at[b, pl.ds(KH, KH)], gsem[b])

    def _wait_gather(b):
        pltpu.make_async_copy(h_hbm.at[pl.ds(0, KH)],
                              rows_v.at[b, pl.ds(0, KH)], gsem[b]).wait()
        pltpu.make_async_copy(h_hbm.at[pl.ds(0, KH)],
                              rows_v.at[b, pl.ds(KH, KH)], gsem[b]).wait()

    def _start_scatter(b, q):
        pltpu.async_copy(rows_v.at[b], acc_sh.at[ei_v[q].at[1]], ssem[b],
                         add=True)

    def _wait_scatter(b):
        pltpu.make_async_copy(rows_v.at[b], acc_sh.at[ei_v0.at[1]],
                              ssem[b]).wait()

    _start_idx(0, 0)
    _start_idx(1, 1)
    _start_idx(2, 2)

    def _zero_row(i, _):
        for c in range(D // 16):
            stage_v[i, pl.ds(c * 16, 16)] = jnp.zeros((16,), jnp.float32)
        return 0
    lax.fori_loop(0, CP, _zero_row, 0)
    _wait_idx(0)
    _start_gather(0, 0)
    JB = (NBLK + NS - 1) // NS
    for j in range(JB):
        blk = j * NS + sid
        @pl.when(blk < NBLK)
        def _():
            pltpu.async_copy(stage_v, acc_sh.at[pl.ds(blk * CP, CP)], zsem)
    for j in range(JB):
        blk = j * NS + sid
        @pl.when(blk < NBLK)
        def _():
            pltpu.make_async_copy(stage_v, acc_sh.at[pl.ds(blk * CP, CP)],
                                  zsem).wait()
    plsc.subcore_barrier()

    @pl.when(wid < NXTRA)
    def _():
        off = (NCHUNK * NW + wid) * K
        pltpu.sync_copy(ei_hbm.at[:, pl.ds(off, K)], ei_vx)
        srcx = ei_vx.at[0]
        pltpu.async_copy(h_hbm.at[srcx.at[pl.ds(0, KH)]],
                         rows_v.at[1, pl.ds(0, KH)], gsem[1])
        pltpu.async_copy(h_hbm.at[srcx.at[pl.ds(KH, KH)]],
                         rows_v.at[1, pl.ds(KH, KH)], gsem[1])
        pltpu.make_async_copy(h_hbm.at[pl.ds(0, KH)],
                              rows_v.at[1, pl.ds(0, KH)], gsem[1]).wait()
        pltpu.make_async_copy(h_hbm.at[pl.ds(0, KH)],
                              rows_v.at[1, pl.ds(KH, KH)], gsem[1]).wait()
        pltpu.sync_copy(rows_v.at[1], acc_sh.at[ei_vx.at[1]], add=True)

    def _quad(Q, _):
        for u in range(4):
            ci = Q * 4 + u
            b, nxt, q = u % 2, 1 - u % 2, u
            _wait_gather(b)
            _start_scatter(b, q)
            _wait_idx((q + 1) % 4)
            if u == 0:
                @pl.when(Q >= 1)
                def _():
                    _wait_scatter(nxt)
            else:
                _wait_scatter(nxt)
            _start_gather(nxt, (q + 1) % 4)

            @pl.when(ci + 3 < NCHUNK)
            def _():
                _start_idx(ci + 3, (q + 3) % 4)
        return 0
    lax.fori_loop(0, (NCHUNK - 2) // 4, _quad, 0)

    _wait_gather(0)
    _start_scatter(0, (NCHUNK - 2) % 4)
    _wait_idx((NCHUNK - 1) % 4)
    _wait_scatter(1)
    _start_gather(1, (NCHUNK - 1) % 4)
    _wait_gather(1)
    _start_scatter(1, (NCHUNK - 1) % 4)
    _wait_scatter(0)
    _wait_scatter(1)
    plsc.subcore_barrier()

    for j in range(JB + 2):
        if j >= 2:
            blkw = (j - 2) * NS + sid
            @pl.when(blkw < NBLK)
            def _():
                pltpu.make_async_copy(
                    rows_v.at[j % 2, pl.ds(0, CP)],
                    out_hbm.at[cid, pl.ds(blkw * CP, CP)],
                    gsem[j % 2]).wait()
        if j < JB:
            blk = j * NS + sid
            @pl.when(blk < NBLK)
            def _():
                r0 = blk * CP
                pltpu.sync_copy(acc_sh.at[pl.ds(r0, CP)],
                                rows_v.at[j % 2, pl.ds(0, CP)])
                pltpu.async_copy(rows_v.at[j % 2, pl.ds(0, CP)],
                                 out_hbm.at[cid, pl.ds(r0, CP)],
                                 gsem[j % 2])


def _edge_scatter(h, ei):
    mesh = plsc.VectorSubcoreMesh(core_axis_name="c", subcore_axis_name="s")
    kern = pl.kernel(
        _scatter_body,
        out_type=jax.ShapeDtypeStruct((NC, N_NODES, D), jnp.float32),
        mesh=mesh,
        scratch_types=[
            pltpu.VMEM_SHARED((N_NODES, D), jnp.float32),
            pltpu.VMEM((2, K), jnp.int32),
            pltpu.VMEM((2, K), jnp.int32),
            pltpu.VMEM((2, K), jnp.int32),
            pltpu.VMEM((2, K), jnp.int32),
            pltpu.VMEM((2, K), jnp.int32),
            pltpu.VMEM((2, K, D), jnp.float32),
            pltpu.VMEM((CP, D), jnp.float32),
        ] + [pltpu.SemaphoreType.DMA] * 9,
    )
    return kern(h, ei)


def kernel(node_feats, edge_index, W1, b1, W2, b2):
    b1r = b1.reshape(1, D)
    b2r = b2.reshape(1, D)
    h = _relu_linear(node_feats, W1, b1r)
    partials = _edge_scatter(h, edge_index)
    return _combine(partials, W2, b2r)
